# baseline (speedup 1.0000x reference)
import functools

import jax
import jax.numpy as jnp
from jax import lax
from jax.experimental import pallas as pl
from jax.experimental.pallas import tpu as pltpu
from jax.experimental.pallas import tpu_sc as plsc

_NUM_DIM = 64
_L = 16
_NCHUNK = _NUM_DIM // _L
_BLK = 128

_GATHER_DNUMS = lax.GatherDimensionNumbers(
    offset_dims=(), collapsed_slice_dims=(0,), start_index_map=(0,))


def _rotate(v, s):
    idx = lax.bitwise_and(lax.iota(jnp.int32, _L) + s, _L - 1)
    return lax.gather(v, idx[:, None], dimension_numbers=_GATHER_DNUMS,
                      slice_sizes=(1,),
                      mode=lax.GatherScatterMode.PROMISE_IN_BOUNDS)


def _all_lanes_sum(v):
    for s in (8, 4, 2, 1):
        v = v + _rotate(v, s)
    return v


def _lane_bcast(v, lane):
    idx = jnp.full((_L, 1), lane, jnp.int32)
    return lax.gather(v, idx, dimension_numbers=_GATHER_DNUMS,
                      slice_sizes=(1,),
                      mode=lax.GatherScatterMode.PROMISE_IN_BOUNDS)


def _column(buf_v, col):
    cols = jnp.full((_L,), col, jnp.int32)
    out = []
    for k in range(_NCHUNK):
        rows = lax.iota(jnp.int32, _L) + (k * _L)
        out.append(plsc.load_gather(buf_v, [rows, cols]))
    return out


def _sc_body(x_hbm, y_hbm, r_hbm, ent_Wt_hbm, ent_bt_hbm, rel_Wt_hbm,
             rel_bt_hbm,
             out_hbm,
             idx_v, xw_v, xb_v, yw_v, yb_v, rw_v, rb_v, b2_v, out_v, sem):
    tile0 = (lax.axis_index("c") == 0) & (lax.axis_index("s") == 0)

    @pl.when(tile0)
    def _():
        icp = [
            pltpu.async_copy(x_hbm, idx_v.at[pl.ds(0, 1)], sem),
            pltpu.async_copy(y_hbm, idx_v.at[pl.ds(8, 1)], sem),
            pltpu.async_copy(r_hbm, idx_v.at[pl.ds(16, 1)], sem),
        ]
        for cp in icp:
            cp.wait()
        v0 = idx_v[pl.ds(0, _L)]
        v1 = idx_v[pl.ds(_L, _L)]
        x_i = v0[0]
        y_i = v0[8]
        r_i = v1[0]
        x0 = pl.multiple_of(lax.bitwise_and(x_i, -_BLK), _BLK)
        y0 = pl.multiple_of(lax.bitwise_and(y_i, -_BLK), _BLK)
        r0 = pl.multiple_of(lax.bitwise_and(r_i, -_BLK), _BLK)
        cps = [
            pltpu.async_copy(ent_Wt_hbm.at[:, pl.ds(x0, _BLK)], xw_v, sem),
            pltpu.async_copy(ent_bt_hbm.at[:, pl.ds(x0, _BLK)], xb_v, sem),
            pltpu.async_copy(ent_Wt_hbm.at[:, pl.ds(y0, _BLK)], yw_v, sem),
            pltpu.async_copy(ent_bt_hbm.at[:, pl.ds(y0, _BLK)], yb_v, sem),
            pltpu.async_copy(rel_Wt_hbm.at[:, pl.ds(r0, _BLK)], rw_v, sem),
            pltpu.async_copy(rel_bt_hbm.at[:, pl.ds(r0, _BLK)], rb_v, sem),
        ]
        for cp in cps:
            cp.wait()

        xc = lax.bitwise_and(x_i, _BLK - 1)
        yc = lax.bitwise_and(y_i, _BLK - 1)
        rc = lax.bitwise_and(r_i, _BLK - 1)
        xw = _column(xw_v, xc)
        xb = _column(xb_v, xc)
        yw = _column(yw_v, yc)
        yb = _column(yb_v, yc)
        rw = _column(rw_v, rc)
        rb = _column(rb_v, rc)
        a = [xw[k] + xb[k] for k in range(_NCHUNK)]
        b = [yw[k] + yb[k] for k in range(_NCHUNK)]
        rv = [rw[k] + rb[k] for k in range(_NCHUNK)]

        for k in range(_NCHUNK):
            b2_v[pl.ds(k * _L, _L)] = b[k]
            b2_v[pl.ds(_NUM_DIM + k * _L, _L)] = b[k]

        c = [jnp.zeros((_L,), jnp.float32) for _ in range(_NCHUNK)]
        for mk in range(_NCHUNK):
            rr_chunk = lax.rev(rv[_NCHUNK - 1 - mk], (0,))
            for lane in range(_L):
                m = mk * _L + lane
                w = _lane_bcast(rr_chunk, lane)
                for k in range(_NCHUNK):
                    c[k] = c[k] + w * b2_v[pl.ds(m + 1 + k * _L, _L)]

        d = a[0] * c[0]
        for k in range(1, _NCHUNK):
            d = d + a[k] * c[k]
        zv = _all_lanes_sum(d) * (1.0 / _NUM_DIM)
        out_v[...] = 1.0 / (1.0 + jnp.exp(-zv))
        pltpu.sync_copy(out_v, out_hbm)


@jax.jit
def _hole_score(x1, y1, r1, ent_Wt, ent_bt, rel_Wt, rel_bt):
    mesh = plsc.VectorSubcoreMesh(core_axis_name="c", subcore_axis_name="s",
                                  num_cores=1, num_subcores=1)
    run = functools.partial(
        pl.kernel,
        out_type=jax.ShapeDtypeStruct((_L,), jnp.float32),
        mesh=mesh,
        compiler_params=pltpu.CompilerParams(needs_layout_passes=False),
        scratch_types=[
            pltpu.VMEM((2 * _L,), jnp.int32),
            pltpu.VMEM((_NUM_DIM, _BLK), jnp.float32),
            pltpu.VMEM((_NUM_DIM, _BLK), jnp.float32),
            pltpu.VMEM((_NUM_DIM, _BLK), jnp.float32),
            pltpu.VMEM((_NUM_DIM, _BLK), jnp.float32),
            pltpu.VMEM((_NUM_DIM, _BLK), jnp.float32),
            pltpu.VMEM((_NUM_DIM, _BLK), jnp.float32),
            pltpu.VMEM((2 * _NUM_DIM,), jnp.float32),
            pltpu.VMEM((_L,), jnp.float32),
            pltpu.SemaphoreType.DMA,
        ],
    )(_sc_body)
    return run(x1, y1, r1, ent_Wt, ent_bt, rel_Wt, rel_bt)


def kernel(x, y, r, ent_W, ent_b, rel_W, rel_b):
    x1 = jnp.asarray(x, jnp.int32).reshape(1)
    y1 = jnp.asarray(y, jnp.int32).reshape(1)
    r1 = jnp.asarray(r, jnp.int32).reshape(1)
    out = _hole_score(x1, y1, r1, ent_W.T, ent_b.T, rel_W.T, rel_b.T)
    return out[0]
